# baseline (speedup 1.0000x reference)
import functools

import jax
import jax.numpy as jnp
from jax import lax
from jax.experimental import pallas as pl
from jax.experimental.pallas import tpu as pltpu
from jax.experimental.pallas import tpu_sc as plsc

_LAM = 0.9
_NSLOT = 64
_NB = 32
_NEV = 64
_N = 2 * 512 * 512
_NC = 2
_NSUB = 16
_NW = _NC * _NSUB
_COLS_W = _N // _NW
_CT = 1024
_NT = _COLS_W // _CT
_L = 16
_VPR = _CT // _L
_PAD = _NEV + _L


def _sc_combine(sev_hbm, aev_hbm, mf_hbm, x_hbm, out_hbm,
                sev_v, aev_v, w_v, c_v, acc_v, x_v, sem_mf, sem_x, sem_out):
    wid = lax.axis_index("s") * _NC + lax.axis_index("c")

    pltpu.sync_copy(sev_hbm, sev_v.at[pl.ds(0, _NEV)])
    pltpu.sync_copy(aev_hbm, aev_v.at[pl.ds(0, _NEV)])

    ones = jnp.ones((_L,), jnp.float32)
    for j in range(_PAD // _L):
        c_v[pl.ds(j * _L, _L)] = ones

    lane0 = lax.iota(jnp.int32, _L) == 0
    for t in range(_NEV - 1, -1, -1):
        j, lane = divmod(t, _L)
        blk = pl.ds(j * _L, _L)
        st = sev_v[blk][lane]
        at = aev_v[blk][lane]
        v = c_v[pl.ds(st, _L)]
        sfx = v[0]
        wv = w_v[blk]
        w_v[blk] = jnp.where(lax.iota(jnp.int32, _L) == lane, at * sfx, wv)
        c_v[pl.ds(st, _L)] = jnp.where(lane0, sfx * (1.0 - at), v)

    def tile_body(g, _):
        col = wid * _COLS_W + g * _CT
        cps = []
        for s in range(_NSLOT):
            cps.append(pltpu.async_copy(
                mf_hbm.at[pl.ds(s * _N + col, _CT)], acc_v.at[s], sem_mf))
        for b in range(_NB):
            cps.append(pltpu.async_copy(
                x_hbm.at[pl.ds(b * _N + col, _CT)], x_v.at[b], sem_x))
        for cp in cps:
            cp.wait()

        def scale_body(s, _):
            cs = c_v[pl.ds(s, _L)][0]
            for v in range(_VPR):
                sl = pl.ds(v * _L, _L)
                acc_v[s, sl] = acc_v[s, sl] * cs
            return 0

        lax.fori_loop(0, _NSLOT, scale_body, 0)

        def ev_body(t, _):
            st = sev_v[pl.ds(t, _L)][0]
            wt = w_v[pl.ds(t, _L)][0]
            bt = lax.shift_right_logical(t, 1)
            for v in range(_VPR):
                sl = pl.ds(v * _L, _L)
                acc_v[st, sl] = acc_v[st, sl] + wt * x_v[bt, sl]
            return 0

        lax.fori_loop(0, _NEV, ev_body, 0)

        ocps = []
        for s in range(_NSLOT):
            ocps.append(pltpu.async_copy(
                acc_v.at[s], out_hbm.at[pl.ds(s * _N + col, _CT)], sem_out))
        for cp in ocps:
            cp.wait()
        return 0

    lax.fori_loop(0, _NT, tile_body, 0)


_sc_kernel = functools.partial(
    pl.kernel,
    out_type=jax.ShapeDtypeStruct((_NSLOT * _N,), jnp.float32),
    mesh=plsc.VectorSubcoreMesh(core_axis_name="c", subcore_axis_name="s"),
    scratch_types=[
        pltpu.VMEM((_PAD,), jnp.int32),
        pltpu.VMEM((_PAD,), jnp.float32),
        pltpu.VMEM((_PAD,), jnp.float32),
        pltpu.VMEM((_PAD,), jnp.float32),
        pltpu.VMEM((_NSLOT, _CT), jnp.float32),
        pltpu.VMEM((_NB, _CT), jnp.float32),
        pltpu.SemaphoreType.DMA,
        pltpu.SemaphoreType.DMA,
        pltpu.SemaphoreType.DMA,
    ],
)(_sc_combine)


def kernel(x, means_idx_0, prop_means_idx_0, mean_fields):
    b, ch, h, w = x.shape
    s = mean_fields.shape[0]
    n = ch * h * w
    xf = x.reshape(b * n)
    mf = mean_fields.reshape(s * n)
    i0 = means_idx_0.astype(jnp.int32)
    p0 = prop_means_idx_0.astype(jnp.float32)
    rate = jnp.float32(1.0 - _LAM)
    a_ev = jnp.stack([p0 * rate, (1.0 - p0) * rate], axis=1).reshape(2 * b)
    s_ev = jnp.stack([i0, i0 + 1], axis=1).reshape(2 * b)
    out = _sc_kernel(s_ev, a_ev, mf, xf)
    return out.reshape(s, ch, h, w)


# speedup vs baseline: 1.1391x; 1.1391x over previous
import functools

import jax
import jax.numpy as jnp
from jax import lax
from jax.experimental import pallas as pl
from jax.experimental.pallas import tpu as pltpu
from jax.experimental.pallas import tpu_sc as plsc

_LAM = 0.9
_NSLOT = 64
_NB = 32
_NEV = 64
_N = 2 * 512 * 512
_NC = 2
_NSUB = 16
_NW = _NC * _NSUB
_COLS_W = _N // _NW
_CT = 512
_NT = _COLS_W // _CT
_L = 16
_VPR = _CT // _L
_PAD = _NEV + _L


def _sc_combine(sev_hbm, aev_hbm, mf_hbm, x_hbm, out_hbm,
                sev_v, aev_v, w_v, c_v,
                acc_a, acc_b, x_a, x_b,
                sem_in_a, sem_in_b, sem_out_a, sem_out_b):
    wid = lax.axis_index("s") * _NC + lax.axis_index("c")
    base = wid * _COLS_W

    pltpu.sync_copy(sev_hbm, sev_v.at[pl.ds(0, _NEV)])
    pltpu.sync_copy(aev_hbm, aev_v.at[pl.ds(0, _NEV)])

    ones = jnp.ones((_L,), jnp.float32)
    for j in range(_PAD // _L):
        c_v[pl.ds(j * _L, _L)] = ones

    lane0 = lax.iota(jnp.int32, _L) == 0
    for t in range(_NEV - 1, -1, -1):
        j, lane = divmod(t, _L)
        blk = pl.ds(j * _L, _L)
        st = sev_v[blk][lane]
        at = aev_v[blk][lane]
        v = c_v[pl.ds(st, _L)]
        sfx = v[0]
        wv = w_v[blk]
        w_v[blk] = jnp.where(lax.iota(jnp.int32, _L) == lane, at * sfx, wv)
        c_v[pl.ds(st, _L)] = jnp.where(lane0, sfx * (1.0 - at), v)

    bufs = ((acc_a, x_a, sem_in_a, sem_out_a),
            (acc_b, x_b, sem_in_b, sem_out_b))

    def start_in(g, par):
        acc, xb, sem_in, _ = bufs[par]
        col = base + g * _CT
        pltpu.async_copy(mf_hbm.at[:, pl.ds(col, _CT)], acc, sem_in)
        pltpu.async_copy(x_hbm.at[:, pl.ds(col, _CT)], xb, sem_in)

    def wait_in(g, par):
        acc, xb, sem_in, _ = bufs[par]
        col = base + g * _CT
        pltpu.make_async_copy(mf_hbm.at[:, pl.ds(col, _CT)], acc, sem_in).wait()
        pltpu.make_async_copy(x_hbm.at[:, pl.ds(col, _CT)], xb, sem_in).wait()

    def start_out(g, par):
        acc, _, _, sem_out = bufs[par]
        col = base + g * _CT
        pltpu.async_copy(acc, out_hbm.at[:, pl.ds(col, _CT)], sem_out)

    def wait_out(g, par):
        acc, _, _, sem_out = bufs[par]
        col = base + g * _CT
        pltpu.make_async_copy(acc, out_hbm.at[:, pl.ds(col, _CT)], sem_out).wait()

    def compute(par):
        acc, xb, _, _ = bufs[par]

        def scale_body(s, _):
            cs = c_v[pl.ds(s, _L)][0]
            for v in range(_VPR):
                sl = pl.ds(v * _L, _L)
                acc[s, sl] = acc[s, sl] * cs
            return 0

        lax.fori_loop(0, _NSLOT, scale_body, 0)

        def ev_body(t, _):
            st = sev_v[pl.ds(t, _L)][0]
            wt = w_v[pl.ds(t, _L)][0]
            bt = lax.shift_right_logical(t, 1)
            for v in range(_VPR):
                sl = pl.ds(v * _L, _L)
                acc[st, sl] = acc[st, sl] + wt * xb[bt, sl]
            return 0

        lax.fori_loop(0, _NEV, ev_body, 0)

    start_in(0, 0)
    start_in(1, 1)

    def pair_body(i, _):
        g0 = 2 * i
        wait_in(g0, 0)
        compute(0)
        start_out(g0, 0)
        wait_in(g0 + 1, 1)
        compute(1)
        start_out(g0 + 1, 1)
        wait_out(g0, 0)
        start_in(g0 + 2, 0)
        wait_out(g0 + 1, 1)
        start_in(g0 + 3, 1)
        return 0

    lax.fori_loop(0, _NT // 2 - 1, pair_body, 0)

    g0 = _NT - 2
    wait_in(g0, 0)
    compute(0)
    start_out(g0, 0)
    wait_in(g0 + 1, 1)
    compute(1)
    start_out(g0 + 1, 1)
    wait_out(g0, 0)
    wait_out(g0 + 1, 1)


_sc_kernel = functools.partial(
    pl.kernel,
    out_type=jax.ShapeDtypeStruct((_NSLOT, _N), jnp.float32),
    mesh=plsc.VectorSubcoreMesh(core_axis_name="c", subcore_axis_name="s"),
    scratch_types=[
        pltpu.VMEM((_PAD,), jnp.int32),
        pltpu.VMEM((_PAD,), jnp.float32),
        pltpu.VMEM((_PAD,), jnp.float32),
        pltpu.VMEM((_PAD,), jnp.float32),
        pltpu.VMEM((_NSLOT, _CT), jnp.float32),
        pltpu.VMEM((_NSLOT, _CT), jnp.float32),
        pltpu.VMEM((_NB, _CT), jnp.float32),
        pltpu.VMEM((_NB, _CT), jnp.float32),
        pltpu.SemaphoreType.DMA,
        pltpu.SemaphoreType.DMA,
        pltpu.SemaphoreType.DMA,
        pltpu.SemaphoreType.DMA,
    ],
)(_sc_combine)


def kernel(x, means_idx_0, prop_means_idx_0, mean_fields):
    b, ch, h, w = x.shape
    s = mean_fields.shape[0]
    n = ch * h * w
    xf = x.reshape(b, n)
    mf = mean_fields.reshape(s, n)
    i0 = means_idx_0.astype(jnp.int32)
    p0 = prop_means_idx_0.astype(jnp.float32)
    rate = jnp.float32(1.0 - _LAM)
    a_ev = jnp.stack([p0 * rate, (1.0 - p0) * rate], axis=1).reshape(2 * b)
    s_ev = jnp.stack([i0, i0 + 1], axis=1).reshape(2 * b)
    out = _sc_kernel(s_ev, a_ev, mf, xf)
    return out.reshape(s, ch, h, w)


# speedup vs baseline: 1.1888x; 1.0436x over previous
import functools

import jax
import jax.numpy as jnp
from jax import lax
from jax.experimental import pallas as pl
from jax.experimental.pallas import tpu as pltpu
from jax.experimental.pallas import tpu_sc as plsc

_LAM = 0.9
_NSLOT = 64
_NB = 32
_NEV = 64
_N = 2 * 512 * 512
_NC = 2
_NSUB = 16
_NW = _NC * _NSUB
_COLS_W = _N // _NW
_CT = 512
_NT = _COLS_W // _CT
_L = 16
_VPR = _CT // _L
_PAD = _NEV + _L


def _sc_combine(sev_hbm, aev_hbm, mf_hbm, x_hbm, out_hbm,
                sev_s, aev_s, w_s, c_s, sev_v, aev_v,
                acc_a, acc_b, x_a, x_b,
                sem_in_a, sem_in_b, sem_out_a, sem_out_b):
    wid = lax.axis_index("s") * _NC + lax.axis_index("c")
    base = wid * _COLS_W

    pltpu.sync_copy(sev_hbm, sev_v)
    pltpu.sync_copy(aev_hbm, aev_v)
    for t in range(_NEV):
        j, lane = divmod(t, _L)
        blk = pl.ds(j * _L, _L)
        sev_s[t] = sev_v[blk][lane]
        aev_s[t] = aev_v[blk][lane]

    def init_body(s, _):
        c_s[s] = 1.0
        return 0

    lax.fori_loop(0, _NSLOT, init_body, 0)

    def rec_body(i, _):
        t = _NEV - 1 - i
        st = sev_s[t]
        at = aev_s[t]
        sfx = c_s[st]
        w_s[t] = at * sfx
        c_s[st] = sfx * (1.0 - at)
        return 0

    lax.fori_loop(0, _NEV, rec_body, 0)

    bufs = ((acc_a, x_a, sem_in_a, sem_out_a),
            (acc_b, x_b, sem_in_b, sem_out_b))

    def start_in(g, par):
        acc, xb, sem_in, _ = bufs[par]
        col = base + g * _CT
        pltpu.async_copy(mf_hbm.at[:, pl.ds(col, _CT)], acc, sem_in)
        pltpu.async_copy(x_hbm.at[:, pl.ds(col, _CT)], xb, sem_in)

    def wait_in(g, par):
        acc, xb, sem_in, _ = bufs[par]
        col = base + g * _CT
        pltpu.make_async_copy(mf_hbm.at[:, pl.ds(col, _CT)], acc, sem_in).wait()
        pltpu.make_async_copy(x_hbm.at[:, pl.ds(col, _CT)], xb, sem_in).wait()

    def start_out(g, par):
        acc, _, _, sem_out = bufs[par]
        col = base + g * _CT
        pltpu.async_copy(acc, out_hbm.at[:, pl.ds(col, _CT)], sem_out)

    def wait_out(g, par):
        acc, _, _, sem_out = bufs[par]
        col = base + g * _CT
        pltpu.make_async_copy(acc, out_hbm.at[:, pl.ds(col, _CT)], sem_out).wait()

    def compute(par):
        acc, xb, _, _ = bufs[par]

        def scale_body(s, _):
            cs = c_s[s]
            for v in range(_VPR):
                sl = pl.ds(v * _L, _L)
                acc[s, sl] = acc[s, sl] * cs
            return 0

        lax.fori_loop(0, _NSLOT, scale_body, 0)

        def ev_body(t, _):
            st = sev_s[t]
            wt = w_s[t]
            bt = lax.shift_right_logical(t, 1)
            for v in range(_VPR):
                sl = pl.ds(v * _L, _L)
                acc[st, sl] = acc[st, sl] + wt * xb[bt, sl]
            return 0

        lax.fori_loop(0, _NEV, ev_body, 0)

    start_in(0, 0)
    start_in(1, 1)

    def pair_body(i, _):
        g0 = 2 * i
        wait_in(g0, 0)
        compute(0)
        start_out(g0, 0)
        wait_in(g0 + 1, 1)
        compute(1)
        start_out(g0 + 1, 1)
        wait_out(g0, 0)
        start_in(g0 + 2, 0)
        wait_out(g0 + 1, 1)
        start_in(g0 + 3, 1)
        return 0

    lax.fori_loop(0, _NT // 2 - 1, pair_body, 0)

    g0 = _NT - 2
    wait_in(g0, 0)
    compute(0)
    start_out(g0, 0)
    wait_in(g0 + 1, 1)
    compute(1)
    start_out(g0 + 1, 1)
    wait_out(g0, 0)
    wait_out(g0 + 1, 1)


_sc_kernel = functools.partial(
    pl.kernel,
    out_type=jax.ShapeDtypeStruct((_NSLOT, _N), jnp.float32),
    mesh=plsc.VectorSubcoreMesh(core_axis_name="c", subcore_axis_name="s"),
    scratch_types=[
        pltpu.SMEM((_NEV,), jnp.int32),
        pltpu.SMEM((_NEV,), jnp.float32),
        pltpu.SMEM((_NEV,), jnp.float32),
        pltpu.SMEM((_NSLOT,), jnp.float32),
        pltpu.VMEM((_NEV,), jnp.int32),
        pltpu.VMEM((_NEV,), jnp.float32),
        pltpu.VMEM((_NSLOT, _CT), jnp.float32),
        pltpu.VMEM((_NSLOT, _CT), jnp.float32),
        pltpu.VMEM((_NB, _CT), jnp.float32),
        pltpu.VMEM((_NB, _CT), jnp.float32),
        pltpu.SemaphoreType.DMA,
        pltpu.SemaphoreType.DMA,
        pltpu.SemaphoreType.DMA,
        pltpu.SemaphoreType.DMA,
    ],
)(_sc_combine)


def kernel(x, means_idx_0, prop_means_idx_0, mean_fields):
    b, ch, h, w = x.shape
    s = mean_fields.shape[0]
    n = ch * h * w
    xf = x.reshape(b, n)
    mf = mean_fields.reshape(s, n)
    i0 = means_idx_0.astype(jnp.int32)
    p0 = prop_means_idx_0.astype(jnp.float32)
    rate = jnp.float32(1.0 - _LAM)
    a_ev = jnp.stack([p0 * rate, (1.0 - p0) * rate], axis=1).reshape(2 * b)
    s_ev = jnp.stack([i0, i0 + 1], axis=1).reshape(2 * b)
    out = _sc_kernel(s_ev, a_ev, mf, xf)
    return out.reshape(s, ch, h, w)


# speedup vs baseline: 1.7111x; 1.4393x over previous
import functools

import jax
import jax.numpy as jnp
from jax import lax
from jax.experimental import pallas as pl
from jax.experimental.pallas import tpu as pltpu
from jax.experimental.pallas import tpu_sc as plsc

_LAM = 0.9
_NSLOT = 64
_NB = 32
_NEV = 64
_N = 2 * 512 * 512
_NC = 2
_NSUB = 16
_NW = _NC * _NSUB
_COLS_W = _N // _NW
_CT = 512
_NT = _COLS_W // _CT
_L = 16
_VPR = _CT // _L
_PAD = _NEV + _L


def _sc_combine(sev_hbm, aev_hbm, mf_hbm, x_hbm, out_hbm,
                sev_s, aev_s, w_s, c_s, sev_v, aev_v,
                cnt_s, start_s, cur_s, wb_w, wb_b,
                acc_a, acc_b, x_a, x_b,
                sem_in_a, sem_in_b, sem_out_a, sem_out_b):
    wid = lax.axis_index("s") * _NC + lax.axis_index("c")
    base = wid * _COLS_W

    pltpu.sync_copy(sev_hbm, sev_v)
    pltpu.sync_copy(aev_hbm, aev_v)
    for t in range(_NEV):
        j, lane = divmod(t, _L)
        blk = pl.ds(j * _L, _L)
        sev_s[t] = sev_v[blk][lane]
        aev_s[t] = aev_v[blk][lane]

    def init_body(s, _):
        c_s[s] = 1.0
        return 0

    lax.fori_loop(0, _NSLOT, init_body, 0)

    def rec_body(i, _):
        t = _NEV - 1 - i
        st = sev_s[t]
        at = aev_s[t]
        sfx = c_s[st]
        w_s[t] = at * sfx
        c_s[st] = sfx * (1.0 - at)
        return 0

    lax.fori_loop(0, _NEV, rec_body, 0)

    def zero_body(s, _):
        cnt_s[s] = 0
        return 0

    lax.fori_loop(0, _NSLOT, zero_body, 0)

    def count_body(t, _):
        st = sev_s[t]
        cnt_s[st] = cnt_s[st] + 1
        return 0

    lax.fori_loop(0, _NEV, count_body, 0)

    def prefix_body(s, run):
        start_s[s] = run
        cur_s[s] = run
        return run + cnt_s[s]

    lax.fori_loop(0, _NSLOT, prefix_body, jnp.int32(0))

    def place_body(t, _):
        st = sev_s[t]
        p = cur_s[st]
        wb_w[p] = w_s[t]
        wb_b[p] = lax.shift_right_logical(t, 1)
        cur_s[st] = p + 1
        return 0

    lax.fori_loop(0, _NEV, place_body, 0)

    bufs = ((acc_a, x_a, sem_in_a, sem_out_a),
            (acc_b, x_b, sem_in_b, sem_out_b))

    _HS = _NSLOT // 2

    def start_in(g, par):
        acc, xb, sem_in, _ = bufs[par]
        col = base + g * _CT
        for h in range(2):
            rows = pl.ds(h * _HS, _HS)
            pltpu.async_copy(mf_hbm.at[rows, pl.ds(col, _CT)],
                             acc.at[rows], sem_in)
        pltpu.async_copy(x_hbm.at[:, pl.ds(col, _CT)], xb, sem_in)

    def wait_in(g, par):
        acc, xb, sem_in, _ = bufs[par]
        col = base + g * _CT
        for h in range(2):
            rows = pl.ds(h * _HS, _HS)
            pltpu.make_async_copy(mf_hbm.at[rows, pl.ds(col, _CT)],
                                  acc.at[rows], sem_in).wait()
        pltpu.make_async_copy(x_hbm.at[:, pl.ds(col, _CT)], xb, sem_in).wait()

    def start_out(g, par):
        acc, _, _, sem_out = bufs[par]
        col = base + g * _CT
        for h in range(2):
            rows = pl.ds(h * _HS, _HS)
            pltpu.async_copy(acc.at[rows],
                             out_hbm.at[rows, pl.ds(col, _CT)], sem_out)

    def wait_out(g, par):
        acc, _, _, sem_out = bufs[par]
        col = base + g * _CT
        for h in range(2):
            rows = pl.ds(h * _HS, _HS)
            pltpu.make_async_copy(acc.at[rows],
                                  out_hbm.at[rows, pl.ds(col, _CT)],
                                  sem_out).wait()

    _VG = 8
    _NG = _VPR // _VG

    def compute(par):
        acc, xb, _, _ = bufs[par]

        def slot_body(s, _):
            cs = c_s[s]
            e0 = start_s[s]
            e1 = e0 + cnt_s[s]
            for g in range(_NG):
                sls = [pl.ds((g * _VG + u) * _L, _L) for u in range(_VG)]
                regs = tuple(cs * acc[s, sl] for sl in sls)

                def ev_body(j, rs):
                    wj = wb_w[j]
                    bj = wb_b[j]
                    return tuple(r + wj * xb[bj, sl]
                                 for r, sl in zip(rs, sls))

                regs = lax.fori_loop(e0, e1, ev_body, regs)
                for r, sl in zip(regs, sls):
                    acc[s, sl] = r
            return 0

        lax.fori_loop(0, _NSLOT, slot_body, 0)

    start_in(0, 0)
    start_in(1, 1)

    def pair_body(i, _):
        g0 = 2 * i
        wait_in(g0, 0)
        compute(0)
        start_out(g0, 0)
        wait_in(g0 + 1, 1)
        compute(1)
        start_out(g0 + 1, 1)
        wait_out(g0, 0)
        start_in(g0 + 2, 0)
        wait_out(g0 + 1, 1)
        start_in(g0 + 3, 1)
        return 0

    lax.fori_loop(0, _NT // 2 - 1, pair_body, 0)

    g0 = _NT - 2
    wait_in(g0, 0)
    compute(0)
    start_out(g0, 0)
    wait_in(g0 + 1, 1)
    compute(1)
    start_out(g0 + 1, 1)
    wait_out(g0, 0)
    wait_out(g0 + 1, 1)


_sc_kernel = functools.partial(
    pl.kernel,
    out_type=jax.ShapeDtypeStruct((_NSLOT, _N), jnp.float32),
    mesh=plsc.VectorSubcoreMesh(core_axis_name="c", subcore_axis_name="s"),
    scratch_types=[
        pltpu.SMEM((_NEV,), jnp.int32),
        pltpu.SMEM((_NEV,), jnp.float32),
        pltpu.SMEM((_NEV,), jnp.float32),
        pltpu.SMEM((_NSLOT,), jnp.float32),
        pltpu.VMEM((_NEV,), jnp.int32),
        pltpu.VMEM((_NEV,), jnp.float32),
        pltpu.SMEM((_NSLOT,), jnp.int32),
        pltpu.SMEM((_NSLOT,), jnp.int32),
        pltpu.SMEM((_NSLOT,), jnp.int32),
        pltpu.SMEM((_NEV,), jnp.float32),
        pltpu.SMEM((_NEV,), jnp.int32),
        pltpu.VMEM((_NSLOT, _CT), jnp.float32),
        pltpu.VMEM((_NSLOT, _CT), jnp.float32),
        pltpu.VMEM((_NB, _CT), jnp.float32),
        pltpu.VMEM((_NB, _CT), jnp.float32),
        pltpu.SemaphoreType.DMA,
        pltpu.SemaphoreType.DMA,
        pltpu.SemaphoreType.DMA,
        pltpu.SemaphoreType.DMA,
    ],
)(_sc_combine)


def kernel(x, means_idx_0, prop_means_idx_0, mean_fields):
    b, ch, h, w = x.shape
    s = mean_fields.shape[0]
    n = ch * h * w
    xf = x.reshape(b, n)
    mf = mean_fields.reshape(s, n)
    i0 = means_idx_0.astype(jnp.int32)
    p0 = prop_means_idx_0.astype(jnp.float32)
    rate = jnp.float32(1.0 - _LAM)
    a_ev = jnp.stack([p0 * rate, (1.0 - p0) * rate], axis=1).reshape(2 * b)
    s_ev = jnp.stack([i0, i0 + 1], axis=1).reshape(2 * b)
    out = _sc_kernel(s_ev, a_ev, mf, xf)
    return out.reshape(s, ch, h, w)


# speedup vs baseline: 1.8049x; 1.0548x over previous
import functools

import jax
import jax.numpy as jnp
from jax import lax
from jax.experimental import pallas as pl
from jax.experimental.pallas import tpu as pltpu
from jax.experimental.pallas import tpu_sc as plsc

_LAM = 0.9
_NSLOT = 64
_NB = 32
_NEV = 64
_N = 2 * 512 * 512
_NC = 2
_NSUB = 16
_NW = _NC * _NSUB
_COLS_W = _N // _NW
_CT = 512
_NT = _COLS_W // _CT
_L = 16
_VPR = _CT // _L
_PAD = _NEV + _L


def _sc_combine(sev_hbm, aev_hbm, mf_hbm, x_hbm, out_hbm,
                sev_s, aev_s, w_s, c_s, sev_v, aev_v,
                cnt_s, start_s, cur_s, wb_w, wb_b, tl_s,
                acc_a, acc_b, x_a, x_b,
                sem_in_a, sem_in_b, sem_out_a, sem_out_b):
    wid = lax.axis_index("s") * _NC + lax.axis_index("c")
    base = wid * _COLS_W

    pltpu.sync_copy(sev_hbm, sev_v)
    pltpu.sync_copy(aev_hbm, aev_v)
    for t in range(_NEV):
        j, lane = divmod(t, _L)
        blk = pl.ds(j * _L, _L)
        sev_s[t] = sev_v[blk][lane]
        aev_s[t] = aev_v[blk][lane]

    def init_body(s, _):
        c_s[s] = 1.0
        return 0

    lax.fori_loop(0, _NSLOT, init_body, 0)

    def rec_body(i, _):
        t = _NEV - 1 - i
        st = sev_s[t]
        at = aev_s[t]
        sfx = c_s[st]
        w_s[t] = at * sfx
        c_s[st] = sfx * (1.0 - at)
        return 0

    lax.fori_loop(0, _NEV, rec_body, 0)

    def zero_body(s, _):
        cnt_s[s] = 0
        return 0

    lax.fori_loop(0, _NSLOT, zero_body, 0)

    def count_body(t, _):
        st = sev_s[t]
        cnt_s[st] = cnt_s[st] + 1
        return 0

    lax.fori_loop(0, _NEV, count_body, 0)

    def prefix_body(s, run):
        start_s[s] = run
        cur_s[s] = run
        return run + cnt_s[s]

    lax.fori_loop(0, _NSLOT, prefix_body, jnp.int32(0))

    def place_body(t, _):
        st = sev_s[t]
        p = cur_s[st]
        wb_w[p] = w_s[t]
        wb_b[p] = lax.shift_right_logical(t, 1)
        cur_s[st] = p + 1
        return 0

    lax.fori_loop(0, _NEV, place_body, 0)

    def tl_body(s, m):
        tl_s[m] = s
        return m + jnp.where(cnt_s[s] > 0, 1, 0).astype(jnp.int32)

    n_touched = lax.fori_loop(0, _NSLOT, tl_body, jnp.int32(0))

    bufs = ((acc_a, x_a, sem_in_a, sem_out_a),
            (acc_b, x_b, sem_in_b, sem_out_b))

    _HS = _NSLOT // 2

    def start_in(g, par):
        acc, xb, sem_in, _ = bufs[par]
        col = base + g * _CT
        for h in range(2):
            rows = pl.ds(h * _HS, _HS)
            pltpu.async_copy(mf_hbm.at[rows, pl.ds(col, _CT)],
                             acc.at[rows], sem_in)
        pltpu.async_copy(x_hbm.at[:, pl.ds(col, _CT)], xb, sem_in)

    def wait_in(g, par):
        acc, xb, sem_in, _ = bufs[par]
        col = base + g * _CT
        for h in range(2):
            rows = pl.ds(h * _HS, _HS)
            pltpu.make_async_copy(mf_hbm.at[rows, pl.ds(col, _CT)],
                                  acc.at[rows], sem_in).wait()
        pltpu.make_async_copy(x_hbm.at[:, pl.ds(col, _CT)], xb, sem_in).wait()

    def start_out(g, par):
        acc, _, _, sem_out = bufs[par]
        col = base + g * _CT
        for h in range(2):
            rows = pl.ds(h * _HS, _HS)
            pltpu.async_copy(acc.at[rows],
                             out_hbm.at[rows, pl.ds(col, _CT)], sem_out)

    def wait_out(g, par):
        acc, _, _, sem_out = bufs[par]
        col = base + g * _CT
        for h in range(2):
            rows = pl.ds(h * _HS, _HS)
            pltpu.make_async_copy(acc.at[rows],
                                  out_hbm.at[rows, pl.ds(col, _CT)],
                                  sem_out).wait()

    _VG = 8
    _NG = _VPR // _VG

    def compute(par):
        acc, xb, _, _ = bufs[par]

        def slot_body(m, _):
            s = tl_s[m]
            cs = c_s[s]
            e0 = start_s[s]
            e1 = e0 + cnt_s[s]
            for g in range(_NG):
                sls = [pl.ds((g * _VG + u) * _L, _L) for u in range(_VG)]
                regs = tuple(cs * acc[s, sl] for sl in sls)

                def ev_body(j, rs):
                    wj = wb_w[j]
                    bj = wb_b[j]
                    return tuple(r + wj * xb[bj, sl]
                                 for r, sl in zip(rs, sls))

                regs = lax.fori_loop(e0, e1, ev_body, regs)
                for r, sl in zip(regs, sls):
                    acc[s, sl] = r
            return 0

        lax.fori_loop(0, n_touched, slot_body, 0)

    start_in(0, 0)
    start_in(1, 1)

    def pair_body(i, _):
        g0 = 2 * i
        wait_in(g0, 0)
        compute(0)
        start_out(g0, 0)
        wait_in(g0 + 1, 1)
        compute(1)
        start_out(g0 + 1, 1)
        wait_out(g0, 0)
        start_in(g0 + 2, 0)
        wait_out(g0 + 1, 1)
        start_in(g0 + 3, 1)
        return 0

    lax.fori_loop(0, _NT // 2 - 1, pair_body, 0)

    g0 = _NT - 2
    wait_in(g0, 0)
    compute(0)
    start_out(g0, 0)
    wait_in(g0 + 1, 1)
    compute(1)
    start_out(g0 + 1, 1)
    wait_out(g0, 0)
    wait_out(g0 + 1, 1)


_sc_kernel = functools.partial(
    pl.kernel,
    out_type=jax.ShapeDtypeStruct((_NSLOT, _N), jnp.float32),
    mesh=plsc.VectorSubcoreMesh(core_axis_name="c", subcore_axis_name="s"),
    scratch_types=[
        pltpu.SMEM((_NEV,), jnp.int32),
        pltpu.SMEM((_NEV,), jnp.float32),
        pltpu.SMEM((_NEV,), jnp.float32),
        pltpu.SMEM((_NSLOT,), jnp.float32),
        pltpu.VMEM((_NEV,), jnp.int32),
        pltpu.VMEM((_NEV,), jnp.float32),
        pltpu.SMEM((_NSLOT,), jnp.int32),
        pltpu.SMEM((_NSLOT,), jnp.int32),
        pltpu.SMEM((_NSLOT,), jnp.int32),
        pltpu.SMEM((_NEV,), jnp.float32),
        pltpu.SMEM((_NEV,), jnp.int32),
        pltpu.SMEM((_NSLOT,), jnp.int32),
        pltpu.VMEM((_NSLOT, _CT), jnp.float32),
        pltpu.VMEM((_NSLOT, _CT), jnp.float32),
        pltpu.VMEM((_NB, _CT), jnp.float32),
        pltpu.VMEM((_NB, _CT), jnp.float32),
        pltpu.SemaphoreType.DMA,
        pltpu.SemaphoreType.DMA,
        pltpu.SemaphoreType.DMA,
        pltpu.SemaphoreType.DMA,
    ],
)(_sc_combine)


def kernel(x, means_idx_0, prop_means_idx_0, mean_fields):
    b, ch, h, w = x.shape
    s = mean_fields.shape[0]
    n = ch * h * w
    xf = x.reshape(b, n)
    mf = mean_fields.reshape(s, n)
    i0 = means_idx_0.astype(jnp.int32)
    p0 = prop_means_idx_0.astype(jnp.float32)
    rate = jnp.float32(1.0 - _LAM)
    a_ev = jnp.stack([p0 * rate, (1.0 - p0) * rate], axis=1).reshape(2 * b)
    s_ev = jnp.stack([i0, i0 + 1], axis=1).reshape(2 * b)
    out = _sc_kernel(s_ev, a_ev, mf, xf)
    return out.reshape(s, ch, h, w)
